# baseline (speedup 1.0000x reference)
import functools

import jax
import jax.numpy as jnp
from jax import lax
from jax.experimental import pallas as pl
from jax.experimental.pallas import tpu as pltpu
from jax.experimental.pallas import tpu_sc as plsc

BATCH = 16384
N_FEAT = 39
TOTAL = BATCH * N_FEAT
NUM_WORKERS = 32
CHUNK = TOTAL // NUM_WORKERS
LANES = 16
UNROLL = 8
N_STEPS = CHUNK // (LANES * UNROLL)


def _sc_body(in_hbm, out_hbm, buf):
    wid = lax.axis_index("s") * 2 + lax.axis_index("c")
    base = wid * CHUNK
    pltpu.sync_copy(in_hbm.at[pl.ds(base, CHUNK)], buf)

    iota = lax.iota(jnp.int32, LANES)

    def step(j, carry):
        for k in range(UNROLL):
            off = (j * UNROLL + k) * LANES
            v = buf[pl.ds(off, LANES)]
            col = (base + off + iota) % N_FEAT
            is_cat = col >= 13
            vi = v.astype(jnp.int32).astype(jnp.float32)
            ok = (vi == v) & (v >= 0.0) & (v <= 15.0)
            enc = jnp.where(ok, v + 1.0, 0.0)
            buf[pl.ds(off, LANES)] = jnp.where(is_cat, enc, v)
        return carry

    lax.fori_loop(0, N_STEPS, step, 0)
    pltpu.sync_copy(buf, out_hbm.at[pl.ds(base, CHUNK)])


@jax.jit
def _sc_encode(flat):
    k = pl.kernel(
        _sc_body,
        out_type=jax.ShapeDtypeStruct((TOTAL,), jnp.float32),
        mesh=plsc.VectorSubcoreMesh(core_axis_name="c", subcore_axis_name="s"),
        scratch_types=[pltpu.VMEM((CHUNK,), jnp.float32)],
    )
    return k(flat)


def kernel(inputs):
    flat = inputs.reshape(TOTAL)
    return _sc_encode(flat).reshape(BATCH, N_FEAT)


# speedup vs baseline: 1.0011x; 1.0011x over previous
import jax
import jax.numpy as jnp
from jax import lax
from jax.experimental import pallas as pl
from jax.experimental.pallas import tpu as pltpu
from jax.experimental.pallas import tpu_sc as plsc

BATCH = 16384
N_FEAT = 39
LANES = 16
NUM_WORKERS = 32

SC_ROWS = 512
SC_TOTAL = SC_ROWS * N_FEAT
SC_CHUNK = SC_TOTAL // NUM_WORKERS

TC_ROWS = BATCH - SC_ROWS
TC_BLOCK = 512
TC_GRID = TC_ROWS // TC_BLOCK


def _encode_block(v, col):
    is_cat = col >= 13
    vi = v.astype(jnp.int32).astype(jnp.float32)
    ok = (vi == v) & (v >= 0.0) & (v <= 15.0)
    return jnp.where(is_cat, jnp.where(ok, v + 1.0, 0.0), v)


def _sc_body(in_hbm, out_hbm, buf):
    wid = lax.axis_index("s") * 2 + lax.axis_index("c")
    base = wid * SC_CHUNK
    pltpu.sync_copy(in_hbm.at[pl.ds(base, SC_CHUNK)], buf)
    iota = lax.iota(jnp.int32, LANES)
    for i in range(SC_CHUNK // LANES):
        off = i * LANES
        v = buf[pl.ds(off, LANES)]
        col = (base + off + iota) % N_FEAT
        buf[pl.ds(off, LANES)] = _encode_block(v, col)
    pltpu.sync_copy(buf, out_hbm.at[pl.ds(base, SC_CHUNK)])


def _tc_body(x_ref, o_ref):
    v = x_ref[...]
    col = lax.broadcasted_iota(jnp.int32, v.shape, 1)
    o_ref[...] = _encode_block(v, col)


@jax.jit
def _encode(inputs):
    flat = inputs.reshape(BATCH * N_FEAT)
    sc_k = pl.kernel(
        _sc_body,
        out_type=jax.ShapeDtypeStruct((SC_TOTAL,), jnp.float32),
        mesh=plsc.VectorSubcoreMesh(core_axis_name="c", subcore_axis_name="s"),
        scratch_types=[pltpu.VMEM((SC_CHUNK,), jnp.float32)],
    )
    sc_out = sc_k(flat)

    tc_out = pl.pallas_call(
        _tc_body,
        grid=(TC_GRID,),
        in_specs=[pl.BlockSpec((TC_BLOCK, N_FEAT), lambda i: (i + 1, 0))],
        out_specs=pl.BlockSpec((TC_BLOCK, N_FEAT), lambda i: (i, 0)),
        out_shape=jax.ShapeDtypeStruct((TC_ROWS, N_FEAT), jnp.float32),
    )(inputs)

    return jnp.concatenate([sc_out.reshape(SC_ROWS, N_FEAT), tc_out], axis=0)


def kernel(inputs):
    return _encode(inputs)


# speedup vs baseline: 2.5712x; 2.5682x over previous
import jax
import jax.numpy as jnp
from jax import lax
from jax.experimental import pallas as pl

BATCH = 16384
N_FEAT = 39
TC_BLOCK = 2048
TC_GRID = BATCH // TC_BLOCK


def _encode_block(v, col):
    is_cat = col >= 13
    vi = v.astype(jnp.int32).astype(jnp.float32)
    ok = (vi == v) & (v >= 0.0) & (v <= 15.0)
    return jnp.where(is_cat, jnp.where(ok, v + 1.0, 0.0), v)


def _tc_body(x_ref, o_ref):
    v = x_ref[...]
    col = lax.broadcasted_iota(jnp.int32, v.shape, 1)
    o_ref[...] = _encode_block(v, col)


@jax.jit
def _encode(inputs):
    return pl.pallas_call(
        _tc_body,
        grid=(TC_GRID,),
        in_specs=[pl.BlockSpec((TC_BLOCK, N_FEAT), lambda i: (i, 0))],
        out_specs=pl.BlockSpec((TC_BLOCK, N_FEAT), lambda i: (i, 0)),
        out_shape=jax.ShapeDtypeStruct((BATCH, N_FEAT), jnp.float32),
    )(inputs)


def kernel(inputs):
    return _encode(inputs)
